# Initial kernel scaffold; baseline (speedup 1.0000x reference)
import jax
import jax.numpy as jnp
from jax.experimental import pallas as pl


def kernel(tests, questions, tags, test_types, W_test, W_question, W_tag, W_test_type, W_pos):
    raise NotImplementedError("write your pallas kernel here")


# speedup vs baseline: 2.5725x; 2.5725x over previous
import functools

import jax
import jax.numpy as jnp
from jax import lax
from jax.experimental import pallas as pl
from jax.experimental.pallas import tpu as pltpu
from jax.experimental.pallas import tpu_sc as plsc

B = 4096
SEQ_LEN = 200
N_DIMS = 64

NC = 2
NS = 16
NW = NC * NS

TOK = B * SEQ_LEN
CHUNK = 128
ROWS = TOK // CHUNK
ROWS_PER_W = ROWS // NW
SUPER = 40
N_SUPER = ROWS_PER_W // SUPER


def _body(tests_i, quests_i, tags_i, types_i, w_test, w_quest, w_tag, w_type,
          w_pos, out, it_v, iq_v, ig_v, iy_v, r0, r1, r2, r3, pos_v,
          gsem, osem):
    cid = lax.axis_index("c")
    sid = lax.axis_index("s")
    wid = sid * NC + cid
    row0 = wid * ROWS_PER_W

    pltpu.sync_copy(w_pos, pos_v)

    def super_body(sc, _):
        rbase = row0 + sc * SUPER
        pltpu.sync_copy(tests_i.at[pl.ds(rbase, SUPER)], it_v)
        pltpu.sync_copy(quests_i.at[pl.ds(rbase, SUPER)], iq_v)
        pltpu.sync_copy(tags_i.at[pl.ds(rbase, SUPER)], ig_v)
        pltpu.sync_copy(types_i.at[pl.ds(rbase, SUPER)], iy_v)

        def chunk_body(j, _):
            g = rbase + j
            tok0 = g * CHUNK
            c0 = pltpu.async_copy(w_test.at[it_v.at[j]], r0, gsem)
            c1 = pltpu.async_copy(w_quest.at[iq_v.at[j]], r1, gsem)
            c2 = pltpu.async_copy(w_tag.at[ig_v.at[j]], r2, gsem)
            c3 = pltpu.async_copy(w_type.at[iy_v.at[j]], r3, gsem)
            c0.wait()
            c1.wait()
            c2.wait()
            c3.wait()
            base_mod = lax.rem(tok0, SEQ_LEN)

            def t_body(t, _):
                s_pos = lax.rem(base_mod + t, SEQ_LEN)
                pbase = s_pos * N_DIMS
                for d in range(N_DIMS // 16):
                    off = d * 16
                    acc = (r0[t, pl.ds(off, 16)] + r1[t, pl.ds(off, 16)]
                           + r2[t, pl.ds(off, 16)] + r3[t, pl.ds(off, 16)]
                           + pos_v[pl.ds(pbase + off, 16)])
                    r0[t, pl.ds(off, 16)] = acc
                return 0

            lax.fori_loop(0, CHUNK, t_body, 0)
            pltpu.async_copy(r0, out.at[pl.ds(tok0, CHUNK)], osem).wait()
            return 0

        lax.fori_loop(0, SUPER, chunk_body, 0)
        return 0

    lax.fori_loop(0, N_SUPER, super_body, 0)


@jax.jit
def kernel(tests, questions, tags, test_types, W_test, W_question, W_tag,
           W_test_type, W_pos):
    tests_i = tests.astype(jnp.int32).reshape(ROWS, CHUNK)
    quests_i = questions.astype(jnp.int32).reshape(ROWS, CHUNK)
    tags_i = tags.astype(jnp.int32).reshape(ROWS, CHUNK)
    types_i = test_types.astype(jnp.int32).reshape(ROWS, CHUNK)
    w_pos_flat = W_pos.reshape(SEQ_LEN * N_DIMS)

    mesh = plsc.VectorSubcoreMesh(core_axis_name="c", subcore_axis_name="s",
                                  num_cores=NC, num_subcores=NS)
    run = pl.kernel(
        _body,
        out_type=jax.ShapeDtypeStruct((TOK, N_DIMS), jnp.float32),
        mesh=mesh,
        compiler_params=pltpu.CompilerParams(use_tc_tiling_on_sc=False),
        scratch_types=[
            pltpu.VMEM((SUPER, CHUNK), jnp.int32),
            pltpu.VMEM((SUPER, CHUNK), jnp.int32),
            pltpu.VMEM((SUPER, CHUNK), jnp.int32),
            pltpu.VMEM((SUPER, CHUNK), jnp.int32),
            pltpu.VMEM((CHUNK, N_DIMS), jnp.float32),
            pltpu.VMEM((CHUNK, N_DIMS), jnp.float32),
            pltpu.VMEM((CHUNK, N_DIMS), jnp.float32),
            pltpu.VMEM((CHUNK, N_DIMS), jnp.float32),
            pltpu.VMEM((SEQ_LEN * N_DIMS,), jnp.float32),
            pltpu.SemaphoreType.DMA,
            pltpu.SemaphoreType.DMA,
        ],
    )
    out = run(tests_i, quests_i, tags_i, types_i, W_test, W_question, W_tag,
              W_test_type, w_pos_flat)
    return out.reshape(B, SEQ_LEN, N_DIMS)
